# baseline (speedup 1.0000x reference)
import functools

import jax
import jax.numpy as jnp
import numpy as np
from jax import lax
from jax.experimental import pallas as pl
from jax.experimental.pallas import tpu as pltpu
from jax.experimental.pallas import tpu_sc as plsc

N = 10000
E = 320000
D_IN = 128
EMB = 32
H = 128
L = 3

NC = 2
NS = 16
NW = NC * NS
CHUNK = 128
NCH = 80
EPT = NCH * CHUNK
EPAD = NW * EPT
PADROWS = 240
NPAD = N + PADROWS
ROWS_PT = NPAD // NS

BLK = 2000
GRID = N // BLK

_f32 = jnp.float32
_BN_SCALE = float(1.0 / np.sqrt(1.0 + 1e-5))


_MESH = plsc.VectorSubcoreMesh(core_axis_name="c", subcore_axis_name="s")


NBUF = 2
GRP = 16


def _sc_aggregate_body(rowp, colp, g, zeros, out, idx_r, idx_c, bufs, acc,
                       sems, ssems):
    c = lax.axis_index("c")
    s = lax.axis_index("s")
    wid = c * NS + s
    for z in range(ROWS_PT // 128):
        pltpu.sync_copy(zeros, acc.at[pl.ds(s * ROWS_PT + z * 128, 128)])
    plsc.subcore_barrier()

    def wait_g(b, ch):
        pltpu.make_async_copy(g.at[idx_r.at[ch]], bufs.at[b],
                              sems.at[b]).wait()

    def scat(b, ch):
        pltpu.async_copy(bufs.at[b], acc.at[idx_c.at[ch]], ssems.at[b],
                         add=True)

    def wait_s(b, ch):
        pltpu.make_async_copy(bufs.at[b], acc.at[idx_c.at[ch]],
                              ssems.at[b]).wait()

    def group(grp, carry):
        pltpu.sync_copy(rowp.at[wid].at[pl.ds(grp * GRP, GRP)], idx_r)
        pltpu.sync_copy(colp.at[wid].at[pl.ds(grp * GRP, GRP)], idx_c)
        pltpu.async_copy(g.at[idx_r.at[0]], bufs.at[0], sems.at[0])
        wait_g(0, 0)
        scat(0, 0)
        pltpu.async_copy(g.at[idx_r.at[1]], bufs.at[1], sems.at[1])

        def inner(k, carry2):
            ch = 2 * k + 1
            wait_g(1, ch)
            scat(1, ch)
            wait_s(0, ch - 1)
            pltpu.async_copy(g.at[idx_r.at[ch + 1]], bufs.at[0], sems.at[0])
            wait_g(0, ch + 1)
            scat(0, ch + 1)
            wait_s(1, ch)
            pltpu.async_copy(g.at[idx_r.at[ch + 2]], bufs.at[1], sems.at[1])
            return carry2

        lax.fori_loop(0, GRP // 2 - 1, inner, 0)
        wait_g(1, GRP - 1)
        scat(1, GRP - 1)
        wait_s(0, GRP - 2)
        wait_s(1, GRP - 1)
        return carry

    lax.fori_loop(0, NCH // GRP, group, 0)
    plsc.subcore_barrier()
    pltpu.sync_copy(acc.at[pl.ds(s * ROWS_PT, ROWS_PT)],
                    out.at[c].at[pl.ds(s * ROWS_PT, ROWS_PT)])


def _make_sc_aggregate(interpret=False):
    return functools.partial(
        pl.kernel,
        out_type=jax.ShapeDtypeStruct((NC, NPAD, H), _f32),
        mesh=_MESH,
        scratch_types=[
            pltpu.VMEM((GRP, CHUNK), jnp.int32),
            pltpu.VMEM((GRP, CHUNK), jnp.int32),
            pltpu.VMEM((NBUF, CHUNK, H), _f32),
            pltpu.VMEM_SHARED((NPAD, H), _f32),
            pltpu.SemaphoreType.DMA((NBUF,)),
            pltpu.SemaphoreType.DMA((NBUF,)),
        ],
        interpret=interpret,
    )(_sc_aggregate_body)


def _sc_degree_body(colp, ones, out, idx_c, buf, acc):
    c = lax.axis_index("c")
    s = lax.axis_index("s")
    wid = c * NS + s
    for z in range(ROWS_PT // 128):
        pltpu.sync_copy(ones, acc.at[pl.ds(s * ROWS_PT + z * 128, 128)])
    pltpu.sync_copy(ones, buf)
    plsc.subcore_barrier()

    def group(grp, carry):
        pltpu.sync_copy(colp.at[wid].at[pl.ds(grp * GRP, GRP)], idx_c)

        def inner(ch, carry2):
            pltpu.sync_copy(buf, acc.at[idx_c.at[ch]], add=True)
            return carry2

        lax.fori_loop(0, GRP, inner, 0)
        return carry

    lax.fori_loop(0, NCH // GRP, group, 0)
    plsc.subcore_barrier()
    pltpu.sync_copy(acc.at[pl.ds(s * ROWS_PT, ROWS_PT)],
                    out.at[c].at[pl.ds(s * ROWS_PT, ROWS_PT)])


def _make_sc_degree(interpret=False):
    return functools.partial(
        pl.kernel,
        out_type=jax.ShapeDtypeStruct((NC, NPAD, H), _f32),
        mesh=_MESH,
        scratch_types=[
            pltpu.VMEM((GRP, CHUNK), jnp.int32),
            pltpu.VMEM((CHUNK, H), _f32),
            pltpu.VMEM_SHARED((NPAD, H), _f32),
        ],
        interpret=interpret,
    )(_sc_degree_body)


_sc_aggregate = _make_sc_aggregate()
_sc_degree = _make_sc_degree()


def _enc_body(x_ref, emb_ref, wft_ref, bft_ref, wc_ref, bc_ref,
              w0_ref, h_ref, u_ref):
    xb = jnp.nan_to_num(x_ref[...])
    feat = jnp.dot(xb, wft_ref[...], preferred_element_type=_f32) + bft_ref[...]
    comb = jnp.concatenate([emb_ref[...], feat], axis=1)
    h0 = jnp.maximum(
        jnp.dot(comb, wc_ref[...], preferred_element_type=_f32) + bc_ref[...], 0.0)
    h_ref[...] = h0
    u_ref[...] = jnp.dot(h0, w0_ref[...], preferred_element_type=_f32)


def _encoder(x, emb, wft, bft, wc, bc, w0):
    return pl.pallas_call(
        _enc_body,
        grid=(GRID,),
        in_specs=[
            pl.BlockSpec((BLK, D_IN), lambda i: (i, 0)),
            pl.BlockSpec((BLK, EMB), lambda i: (i, 0)),
            pl.BlockSpec((D_IN, EMB), lambda i: (0, 0)),
            pl.BlockSpec((1, EMB), lambda i: (0, 0)),
            pl.BlockSpec((2 * EMB, H), lambda i: (0, 0)),
            pl.BlockSpec((1, H), lambda i: (0, 0)),
            pl.BlockSpec((H, H), lambda i: (0, 0)),
        ],
        out_specs=[
            pl.BlockSpec((BLK, H), lambda i: (i, 0)),
            pl.BlockSpec((BLK, H), lambda i: (i, 0)),
        ],
        out_shape=[
            jax.ShapeDtypeStruct((N, H), _f32),
            jax.ShapeDtypeStruct((N, H), _f32),
        ],
    )(x, emb, wft, bft, wc, bc, w0)


def _scale_body(degp_ref, u_ref, dinv_ref, g_ref):
    deg = degp_ref[0][:, 0:16] + degp_ref[1][:, 0:16] - 1.0
    dinv = lax.rsqrt(deg)
    dinv_ref[...] = dinv
    g_ref[...] = dinv[:, 0:1] * u_ref[...]


def _scale(degp, u):
    return pl.pallas_call(
        _scale_body,
        grid=(GRID,),
        in_specs=[
            pl.BlockSpec((NC, BLK, H), lambda i: (0, i, 0)),
            pl.BlockSpec((BLK, H), lambda i: (i, 0)),
        ],
        out_specs=[
            pl.BlockSpec((BLK, 16), lambda i: (i, 0)),
            pl.BlockSpec((BLK, H), lambda i: (i, 0)),
        ],
        out_shape=[
            jax.ShapeDtypeStruct((N, 16), _f32),
            jax.ShapeDtypeStruct((N, H), _f32),
        ],
    )(degp, u)


def _upd_body(sp_ref, g_ref, h_ref, dinv_ref, b_ref, gam_ref, bet_ref,
              wn_ref, hn_ref, gn_ref):
    s = sp_ref[0] + sp_ref[1] + g_ref[...]
    dcol = dinv_ref[...][:, 0:1]
    pre = dcol * s + b_ref[...]
    hn = (jnp.maximum(pre, 0.0) * (gam_ref[...] * _BN_SCALE)
          + bet_ref[...] + h_ref[...])
    hn_ref[...] = hn
    gn_ref[...] = dcol * jnp.dot(hn, wn_ref[...], preferred_element_type=_f32)


def _update(sp, g, h, dinv, bi, gam, bet, wn):
    return pl.pallas_call(
        _upd_body,
        grid=(GRID,),
        in_specs=[
            pl.BlockSpec((NC, BLK, H), lambda i: (0, i, 0)),
            pl.BlockSpec((BLK, H), lambda i: (i, 0)),
            pl.BlockSpec((BLK, H), lambda i: (i, 0)),
            pl.BlockSpec((BLK, 16), lambda i: (i, 0)),
            pl.BlockSpec((1, H), lambda i: (0, 0)),
            pl.BlockSpec((1, H), lambda i: (0, 0)),
            pl.BlockSpec((1, H), lambda i: (0, 0)),
            pl.BlockSpec((H, H), lambda i: (0, 0)),
        ],
        out_specs=[
            pl.BlockSpec((BLK, H), lambda i: (i, 0)),
            pl.BlockSpec((BLK, H), lambda i: (i, 0)),
        ],
        out_shape=[
            jax.ShapeDtypeStruct((N, H), _f32),
            jax.ShapeDtypeStruct((N, H), _f32),
        ],
    )(sp, g, h, dinv, bi, gam, bet, wn)


def _fin_body(sp_ref, g_ref, h_ref, dinv_ref, b_ref, gam_ref, bet_ref,
              wl_ref, bl_ref, out_ref):
    s = sp_ref[0] + sp_ref[1] + g_ref[...]
    dcol = dinv_ref[...][:, 0:1]
    pre = dcol * s + b_ref[...]
    hn = (jnp.maximum(pre, 0.0) * (gam_ref[...] * _BN_SCALE)
          + bet_ref[...] + h_ref[...])
    o = jnp.dot(hn, wl_ref[...], preferred_element_type=_f32) + bl_ref[...]
    out_ref[...] = jnp.clip(o, -10.0, 10.0)


def _final(sp, g, h, dinv, bi, gam, bet, wl, bl):
    return pl.pallas_call(
        _fin_body,
        grid=(GRID,),
        in_specs=[
            pl.BlockSpec((NC, BLK, H), lambda i: (0, i, 0)),
            pl.BlockSpec((BLK, H), lambda i: (i, 0)),
            pl.BlockSpec((BLK, H), lambda i: (i, 0)),
            pl.BlockSpec((BLK, 16), lambda i: (i, 0)),
            pl.BlockSpec((1, H), lambda i: (0, 0)),
            pl.BlockSpec((1, H), lambda i: (0, 0)),
            pl.BlockSpec((1, H), lambda i: (0, 0)),
            pl.BlockSpec((H, 1), lambda i: (0, 0)),
            pl.BlockSpec((1, 1), lambda i: (0, 0)),
        ],
        out_specs=pl.BlockSpec((BLK, 1), lambda i: (i, 0)),
        out_shape=jax.ShapeDtypeStruct((N, 1), _f32),
    )(sp, g, h, dinv, bi, gam, bet, wl, bl)


def kernel(x, edge_index, emb_table, W_ft, b_ft, W_c, b_c, conv_W, conv_b,
           gamma, beta, W_lin, b_lin):
    pad = EPAD - E
    pad_r = (jnp.arange(pad, dtype=jnp.int32) % N)
    pad_c = N + (jnp.arange(pad, dtype=jnp.int32) % PADROWS)
    rowp = jnp.concatenate([edge_index[0], pad_r]).reshape(NW, NCH, CHUNK)
    colp = jnp.concatenate([edge_index[1], pad_c]).reshape(NW, NCH, CHUNK)

    zeros = jnp.zeros((128, H), _f32)
    ones = jnp.ones((CHUNK, H), _f32)

    degp = _sc_degree(colp, ones)

    bft = b_ft.reshape(1, EMB)
    bc = b_c.reshape(1, H)
    gam = gamma.reshape(1, H)
    bet = beta.reshape(1, H)

    h, u0 = _encoder(x, emb_table, W_ft, bft, W_c, bc, conv_W[0])
    dinv, g = _scale(degp, u0)
    out = None
    for i in range(L):
        sp = _sc_aggregate(rowp, colp, g, zeros)
        bi = conv_b[i].reshape(1, H)
        if i < L - 1:
            h, g = _update(sp, g, h, dinv, bi, gam, bet, conv_W[i + 1])
        else:
            out = _final(sp, g, h, dinv, bi, gam, bet, W_lin,
                         b_lin.reshape(1, 1))
    return out


# speedup vs baseline: 1.1608x; 1.1608x over previous
import functools

import jax
import jax.numpy as jnp
import numpy as np
from jax import lax
from jax.experimental import pallas as pl
from jax.experimental.pallas import tpu as pltpu
from jax.experimental.pallas import tpu_sc as plsc

N = 10000
E = 320000
D_IN = 128
EMB = 32
H = 128
L = 3

NC = 2
NS = 16
NW = NC * NS
CHUNK = 128
NCH = 80
EPT = NCH * CHUNK
EPAD = NW * EPT
PADROWS = 240
NPAD = N + PADROWS
ROWS_PT = NPAD // NS

BLK = 2000
GRID = N // BLK

_f32 = jnp.float32
_BN_SCALE = float(1.0 / np.sqrt(1.0 + 1e-5))


_MESH = plsc.VectorSubcoreMesh(core_axis_name="c", subcore_axis_name="s")


NBUF = 2
GRP = 40


def _sc_aggregate_body(rowp, colp, g, zeros, out, idx_r, idx_c, bufs, acc,
                       sems):
    c = lax.axis_index("c")
    s = lax.axis_index("s")
    wid = c * NS + s
    for z in range(ROWS_PT // 128):
        pltpu.sync_copy(zeros, acc.at[pl.ds(s * ROWS_PT + z * 128, 128)])
    plsc.subcore_barrier()

    def group(grp, carry):
        pltpu.sync_copy(rowp.at[wid].at[pl.ds(grp * GRP, GRP)], idx_r)
        pltpu.sync_copy(colp.at[wid].at[pl.ds(grp * GRP, GRP)], idx_c)
        for b in range(NBUF):
            pltpu.async_copy(g.at[idx_r.at[b]], bufs.at[b], sems.at[b])

        def inner(k, carry2):
            base = k * NBUF
            for b in range(NBUF):
                ch = base + b
                pltpu.make_async_copy(g.at[idx_r.at[ch]], bufs.at[b],
                                      sems.at[b]).wait()
                pltpu.sync_copy(bufs.at[b], acc.at[idx_c.at[ch]], add=True)

                @pl.when(ch + NBUF < GRP)
                def _():
                    pltpu.async_copy(g.at[idx_r.at[ch + NBUF]], bufs.at[b],
                                     sems.at[b])

            return carry2

        lax.fori_loop(0, GRP // NBUF, inner, 0)
        return carry

    lax.fori_loop(0, NCH // GRP, group, 0)
    plsc.subcore_barrier()
    pltpu.sync_copy(acc.at[pl.ds(s * ROWS_PT, ROWS_PT)],
                    out.at[c].at[pl.ds(s * ROWS_PT, ROWS_PT)])


def _make_sc_aggregate(interpret=False):
    return functools.partial(
        pl.kernel,
        out_type=jax.ShapeDtypeStruct((NC, NPAD, H), _f32),
        mesh=_MESH,
        scratch_types=[
            pltpu.VMEM((GRP, CHUNK), jnp.int32),
            pltpu.VMEM((GRP, CHUNK), jnp.int32),
            pltpu.VMEM((NBUF, CHUNK, H), _f32),
            pltpu.VMEM_SHARED((NPAD, H), _f32),
            pltpu.SemaphoreType.DMA((NBUF,)),
        ],
        interpret=interpret,
    )(_sc_aggregate_body)


def _sc_degree_body(colp, ones, out, idx_c, buf, acc):
    c = lax.axis_index("c")
    s = lax.axis_index("s")
    wid = c * NS + s
    for z in range(ROWS_PT // 128):
        pltpu.sync_copy(ones, acc.at[pl.ds(s * ROWS_PT + z * 128, 128)])
    pltpu.sync_copy(ones, buf)
    plsc.subcore_barrier()

    def group(grp, carry):
        pltpu.sync_copy(colp.at[wid].at[pl.ds(grp * GRP, GRP)], idx_c)

        def inner(ch, carry2):
            pltpu.sync_copy(buf, acc.at[idx_c.at[ch]], add=True)
            return carry2

        lax.fori_loop(0, GRP, inner, 0)
        return carry

    lax.fori_loop(0, NCH // GRP, group, 0)
    plsc.subcore_barrier()
    pltpu.sync_copy(acc.at[pl.ds(s * ROWS_PT, ROWS_PT)],
                    out.at[c].at[pl.ds(s * ROWS_PT, ROWS_PT)])


def _make_sc_degree(interpret=False):
    return functools.partial(
        pl.kernel,
        out_type=jax.ShapeDtypeStruct((NC, NPAD, H), _f32),
        mesh=_MESH,
        scratch_types=[
            pltpu.VMEM((GRP, CHUNK), jnp.int32),
            pltpu.VMEM((CHUNK, H), _f32),
            pltpu.VMEM_SHARED((NPAD, H), _f32),
        ],
        interpret=interpret,
    )(_sc_degree_body)


_sc_aggregate = _make_sc_aggregate()
_sc_degree = _make_sc_degree()


def _enc_body(x_ref, emb_ref, wft_ref, bft_ref, wc_ref, bc_ref,
              w0_ref, h_ref, u_ref):
    xb = jnp.nan_to_num(x_ref[...])
    feat = jnp.dot(xb, wft_ref[...], preferred_element_type=_f32) + bft_ref[...]
    comb = jnp.concatenate([emb_ref[...], feat], axis=1)
    h0 = jnp.maximum(
        jnp.dot(comb, wc_ref[...], preferred_element_type=_f32) + bc_ref[...], 0.0)
    h_ref[...] = h0
    u_ref[...] = jnp.dot(h0, w0_ref[...], preferred_element_type=_f32)


def _encoder(x, emb, wft, bft, wc, bc, w0):
    return pl.pallas_call(
        _enc_body,
        grid=(GRID,),
        in_specs=[
            pl.BlockSpec((BLK, D_IN), lambda i: (i, 0)),
            pl.BlockSpec((BLK, EMB), lambda i: (i, 0)),
            pl.BlockSpec((D_IN, EMB), lambda i: (0, 0)),
            pl.BlockSpec((1, EMB), lambda i: (0, 0)),
            pl.BlockSpec((2 * EMB, H), lambda i: (0, 0)),
            pl.BlockSpec((1, H), lambda i: (0, 0)),
            pl.BlockSpec((H, H), lambda i: (0, 0)),
        ],
        out_specs=[
            pl.BlockSpec((BLK, H), lambda i: (i, 0)),
            pl.BlockSpec((BLK, H), lambda i: (i, 0)),
        ],
        out_shape=[
            jax.ShapeDtypeStruct((N, H), _f32),
            jax.ShapeDtypeStruct((N, H), _f32),
        ],
    )(x, emb, wft, bft, wc, bc, w0)


def _scale_body(degp_ref, u_ref, dinv_ref, g_ref):
    deg = degp_ref[0][:, 0:16] + degp_ref[1][:, 0:16] - 1.0
    dinv = lax.rsqrt(deg)
    dinv_ref[...] = dinv
    g_ref[...] = dinv[:, 0:1] * u_ref[...]


def _scale(degp, u):
    return pl.pallas_call(
        _scale_body,
        grid=(GRID,),
        in_specs=[
            pl.BlockSpec((NC, BLK, H), lambda i: (0, i, 0)),
            pl.BlockSpec((BLK, H), lambda i: (i, 0)),
        ],
        out_specs=[
            pl.BlockSpec((BLK, 16), lambda i: (i, 0)),
            pl.BlockSpec((BLK, H), lambda i: (i, 0)),
        ],
        out_shape=[
            jax.ShapeDtypeStruct((N, 16), _f32),
            jax.ShapeDtypeStruct((N, H), _f32),
        ],
    )(degp, u)


def _upd_body(sp_ref, g_ref, h_ref, dinv_ref, b_ref, gam_ref, bet_ref,
              wn_ref, hn_ref, gn_ref):
    s = sp_ref[0] + sp_ref[1] + g_ref[...]
    dcol = dinv_ref[...][:, 0:1]
    pre = dcol * s + b_ref[...]
    hn = (jnp.maximum(pre, 0.0) * (gam_ref[...] * _BN_SCALE)
          + bet_ref[...] + h_ref[...])
    hn_ref[...] = hn
    gn_ref[...] = dcol * jnp.dot(hn, wn_ref[...], preferred_element_type=_f32)


def _update(sp, g, h, dinv, bi, gam, bet, wn):
    return pl.pallas_call(
        _upd_body,
        grid=(GRID,),
        in_specs=[
            pl.BlockSpec((NC, BLK, H), lambda i: (0, i, 0)),
            pl.BlockSpec((BLK, H), lambda i: (i, 0)),
            pl.BlockSpec((BLK, H), lambda i: (i, 0)),
            pl.BlockSpec((BLK, 16), lambda i: (i, 0)),
            pl.BlockSpec((1, H), lambda i: (0, 0)),
            pl.BlockSpec((1, H), lambda i: (0, 0)),
            pl.BlockSpec((1, H), lambda i: (0, 0)),
            pl.BlockSpec((H, H), lambda i: (0, 0)),
        ],
        out_specs=[
            pl.BlockSpec((BLK, H), lambda i: (i, 0)),
            pl.BlockSpec((BLK, H), lambda i: (i, 0)),
        ],
        out_shape=[
            jax.ShapeDtypeStruct((N, H), _f32),
            jax.ShapeDtypeStruct((N, H), _f32),
        ],
    )(sp, g, h, dinv, bi, gam, bet, wn)


def _fin_body(sp_ref, g_ref, h_ref, dinv_ref, b_ref, gam_ref, bet_ref,
              wl_ref, bl_ref, out_ref):
    s = sp_ref[0] + sp_ref[1] + g_ref[...]
    dcol = dinv_ref[...][:, 0:1]
    pre = dcol * s + b_ref[...]
    hn = (jnp.maximum(pre, 0.0) * (gam_ref[...] * _BN_SCALE)
          + bet_ref[...] + h_ref[...])
    o = jnp.dot(hn, wl_ref[...], preferred_element_type=_f32) + bl_ref[...]
    out_ref[...] = jnp.clip(o, -10.0, 10.0)


def _final(sp, g, h, dinv, bi, gam, bet, wl, bl):
    return pl.pallas_call(
        _fin_body,
        grid=(GRID,),
        in_specs=[
            pl.BlockSpec((NC, BLK, H), lambda i: (0, i, 0)),
            pl.BlockSpec((BLK, H), lambda i: (i, 0)),
            pl.BlockSpec((BLK, H), lambda i: (i, 0)),
            pl.BlockSpec((BLK, 16), lambda i: (i, 0)),
            pl.BlockSpec((1, H), lambda i: (0, 0)),
            pl.BlockSpec((1, H), lambda i: (0, 0)),
            pl.BlockSpec((1, H), lambda i: (0, 0)),
            pl.BlockSpec((H, 1), lambda i: (0, 0)),
            pl.BlockSpec((1, 1), lambda i: (0, 0)),
        ],
        out_specs=pl.BlockSpec((BLK, 1), lambda i: (i, 0)),
        out_shape=jax.ShapeDtypeStruct((N, 1), _f32),
    )(sp, g, h, dinv, bi, gam, bet, wl, bl)


def kernel(x, edge_index, emb_table, W_ft, b_ft, W_c, b_c, conv_W, conv_b,
           gamma, beta, W_lin, b_lin):
    pad = EPAD - E
    pad_r = (jnp.arange(pad, dtype=jnp.int32) % N)
    pad_c = N + (jnp.arange(pad, dtype=jnp.int32) % PADROWS)
    rowp = jnp.concatenate([edge_index[0], pad_r]).reshape(NW, NCH, CHUNK)
    colp = jnp.concatenate([edge_index[1], pad_c]).reshape(NW, NCH, CHUNK)

    zeros = jnp.zeros((128, H), _f32)
    ones = jnp.ones((CHUNK, H), _f32)

    degp = _sc_degree(colp, ones)

    bft = b_ft.reshape(1, EMB)
    bc = b_c.reshape(1, H)
    gam = gamma.reshape(1, H)
    bet = beta.reshape(1, H)

    h, u0 = _encoder(x, emb_table, W_ft, bft, W_c, bc, conv_W[0])
    dinv, g = _scale(degp, u0)
    out = None
    for i in range(L):
        sp = _sc_aggregate(rowp, colp, g, zeros)
        bi = conv_b[i].reshape(1, H)
        if i < L - 1:
            h, g = _update(sp, g, h, dinv, bi, gam, bet, conv_W[i + 1])
        else:
            out = _final(sp, g, h, dinv, bi, gam, bet, W_lin,
                         b_lin.reshape(1, 1))
    return out
